# Initial kernel scaffold; baseline (speedup 1.0000x reference)
import jax
import jax.numpy as jnp
from jax.experimental import pallas as pl


def kernel(tokens, table):
    raise NotImplementedError("write your pallas kernel here")


# speedup vs baseline: 2.7848x; 2.7848x over previous
import functools

import jax
import jax.numpy as jnp
from jax import lax
from jax.experimental import pallas as pl
from jax.experimental.pallas import tpu as pltpu
from jax.experimental.pallas import tpu_sc as plsc

BPE = 43
EMBED = 320
LANES = 16
M = 16
NSL = EMBED // LANES
GROUPS = (8, 8, 8, 8, 8, 3)


def _make_sc_kernel(n_blocks, n_workers, vocab):
    blocks_per_w = n_blocks // n_workers
    mesh = plsc.VectorSubcoreMesh(core_axis_name="c", subcore_axis_name="s")
    inv = jnp.float32(1.0 / BPE)

    @functools.partial(
        pl.kernel,
        mesh=mesh,
        out_type=jax.ShapeDtypeStruct((n_blocks, M, EMBED), jnp.float32),
        scratch_types=[
            pltpu.VMEM((BPE, M), jnp.int32),
            pltpu.VMEM((8, M, EMBED), jnp.float32),
            pltpu.VMEM((M, EMBED), jnp.float32),
            pltpu.SemaphoreType.DMA,
        ],
        compiler_params=pltpu.CompilerParams(use_tc_tiling_on_sc=False),
    )
    def k(idx_hbm, table_hbm, out_hbm, idx_v, buf, acc, sem):
        wid = lax.axis_index("s") * 2 + lax.axis_index("c")

        def block_body(blk, carry):
            gblk = wid * blocks_per_w + blk
            pltpu.sync_copy(idx_hbm.at[gblk], idx_v)

            t0 = 0
            for gi, glen in enumerate(GROUPS):
                handles = []
                for j in range(glen):
                    handles.append(
                        pltpu.async_copy(
                            table_hbm.at[idx_v.at[t0 + j]], buf.at[j], sem
                        )
                    )
                for h in handles:
                    h.wait()

                first = gi == 0
                last = gi == len(GROUPS) - 1

                def accum_m(m, c):
                    for d in range(NSL):
                        sl = pl.ds(d * LANES, LANES)
                        s = buf[0, m, sl]
                        for j in range(1, glen):
                            s = s + buf[j, m, sl]
                        if not first:
                            s = s + acc[m, sl]
                        if last:
                            s = s * inv
                        acc[m, sl] = s
                    return c

                lax.fori_loop(0, M, accum_m, 0, unroll=False)
                t0 += glen

            pltpu.sync_copy(acc, out_hbm.at[gblk])
            return carry

        lax.fori_loop(0, blocks_per_w, block_body, 0, unroll=False)

    return k


def kernel(tokens, table):
    seq_total, batch = tokens.shape
    n_chunks = seq_total // BPE
    vocab = table.shape[0]
    n_blocks = (n_chunks * batch) // M
    idx = (
        tokens.reshape(n_chunks, BPE, batch // M, M)
        .transpose(0, 2, 1, 3)
        .reshape(n_blocks, BPE, M)
    )
    out = _make_sc_kernel(n_blocks, 32, vocab)(idx, table)
    return out.reshape(n_chunks, batch, EMBED)


# speedup vs baseline: 3.6200x; 1.2999x over previous
import functools

import jax
import jax.numpy as jnp
from jax import lax
from jax.experimental import pallas as pl
from jax.experimental.pallas import tpu as pltpu
from jax.experimental.pallas import tpu_sc as plsc

BPE = 43
EMBED = 320
LANES = 16
M = 16
NSL = EMBED // LANES
STAGES = ((0, 8), (8, 8), (16, 8), (24, 8), (32, 8), (40, 3))
N_WORKERS = 32


def _make_sc_kernel(n_blocks):
    bpw = n_blocks // N_WORKERS
    mesh = plsc.VectorSubcoreMesh(core_axis_name="c", subcore_axis_name="s")
    inv = jnp.float32(1.0 / BPE)

    @functools.partial(
        pl.kernel,
        mesh=mesh,
        out_type=jax.ShapeDtypeStruct((n_blocks, M, EMBED), jnp.float32),
        scratch_types=[
            pltpu.VMEM((bpw * BPE * M,), jnp.int32),
            pltpu.VMEM((8 * M, EMBED), jnp.float32),
            pltpu.VMEM((8 * M, EMBED), jnp.float32),
            pltpu.VMEM((M, EMBED), jnp.float32),
            pltpu.VMEM((M, EMBED), jnp.float32),
            pltpu.SemaphoreType.DMA,
            pltpu.SemaphoreType.DMA,
        ],
        compiler_params=pltpu.CompilerParams(use_tc_tiling_on_sc=False),
    )
    def k(idx_hbm, table_hbm, out_hbm, idx_all, buf_a, buf_b, acc_e, acc_o,
          sem_a, sem_b):
        wid = lax.axis_index("s") * 2 + lax.axis_index("c")
        base = wid * bpw
        blk_words = BPE * M
        pltpu.sync_copy(idx_hbm.at[pl.ds(base * blk_words, bpw * blk_words)],
                        idx_all)

        bufs = (buf_a, buf_b)
        sems = (sem_a, sem_b)

        def gather_desc(blk, sg):
            t0, glen = STAGES[sg % 6]
            return pltpu.make_async_copy(
                table_hbm.at[idx_all.at[pl.ds(blk * blk_words + t0 * M,
                                              glen * M)]],
                bufs[sg % 2].at[pl.ds(0, glen * M)],
                sems[sg % 2],
            )

        def accum(buf, acc, glen, first, last):
            def body(m, c):
                for d in range(NSL):
                    sl = pl.ds(d * LANES, LANES)
                    s = buf[m, sl]
                    for j in range(1, glen):
                        s = s + buf[j * M + m, sl]
                    if not first:
                        s = s + acc[m, sl]
                    if last:
                        s = s * inv
                    acc[m, sl] = s
                return c

            lax.fori_loop(0, M, body, 0, unroll=False)

        def pair_body(kk, carry):
            e = kk * 2
            for half, acc in ((0, acc_e), (1, acc_o)):
                blk = e + half
                for s in range(6):
                    sg = half * 6 + s
                    if sg < 11:
                        gather_desc(blk + (1 if s == 5 else 0), sg + 1).start()
                    else:
                        @pl.when(kk < bpw // 2 - 1)
                        def _():
                            gather_desc(blk + 1, 0).start()
                    gather_desc(blk, sg).wait()
                    accum(bufs[sg % 2], acc, STAGES[s][1],
                          first=(s == 0), last=(s == 5))
                pltpu.sync_copy(acc, out_hbm.at[base + blk])
            return carry

        gather_desc(0, 0).start()
        lax.fori_loop(0, bpw // 2, pair_body, 0, unroll=False)

    return k


def kernel(tokens, table):
    seq_total, batch = tokens.shape
    n_chunks = seq_total // BPE
    n_blocks = (n_chunks * batch) // M
    idx = (
        tokens.reshape(n_chunks, BPE, batch // M, M)
        .transpose(0, 2, 1, 3)
        .reshape(n_blocks * BPE * M)
    )
    out = _make_sc_kernel(n_blocks)(idx, table)
    return out.reshape(n_chunks, batch, EMBED)
